# Initial kernel scaffold; baseline (speedup 1.0000x reference)
import jax
import jax.numpy as jnp
from jax.experimental import pallas as pl


def kernel(p, q, x, W1, b1, g1, be1, W2, b2, g2, be2):
    raise NotImplementedError("write your pallas kernel here")


# speedup vs baseline: 32.3385x; 32.3385x over previous
import functools

import jax
import jax.numpy as jnp
from jax import lax
from jax.experimental import pallas as pl

B, N, M = 2, 2048, 8192
C_IN, C1, C2 = 64, 64, 64
K = 3
TM = 256
NB = M // TM


def _knn_interp_body(q_ref, pT_ref, xT_ref, out_ref):
    qt = q_ref[0]
    qx, qy, qz = qt[:, 0:1], qt[:, 1:2], qt[:, 2:3]
    pt = pT_ref[0]
    px, py, pz = pt[0:1, :], pt[1:2, :], pt[2:3, :]
    dx = qx - px
    dy = qy - py
    dz = qz - pz
    d = dx * dx + dy * dy + dz * dz
    iota = lax.broadcasted_iota(jnp.int32, (TM, N), 1)

    vals, idxs = [], []
    for k in range(K):
        m = jnp.min(d, axis=1, keepdims=True)
        i = jnp.min(jnp.where(d == m, iota, N), axis=1, keepdims=True)
        vals.append(m)
        idxs.append(i)
        if k < K - 1:
            d = jnp.where(iota == i, jnp.inf, d)

    ws = [1.0 / jnp.maximum(v, 1e-10) for v in vals]
    wsum = ws[0] + ws[1] + ws[2]
    ws = [w / wsum for w in ws]

    s = jnp.zeros((TM, N), jnp.float32)
    for w, i in zip(ws, idxs):
        s = s + jnp.where(iota == i, w, 0.0)
    out_ref[0] = lax.dot_general(
        s, xT_ref[0], (((1,), (0,)), ((), ())),
        preferred_element_type=jnp.float32)


def _mlp_body(h_ref, W1_ref, b1_ref, g1_ref, be1_ref,
              W2_ref, b2_ref, g2_ref, be2_ref, out_ref):
    h = h_ref[...]

    def layer(h, W_ref, b_ref, g_ref, be_ref):
        u = lax.dot_general(h, W_ref[...], (((1,), (1,)), ((), ())),
                            preferred_element_type=jnp.float32)
        u = u + b_ref[...]
        mu = jnp.mean(u, axis=0, keepdims=True)
        var = jnp.mean((u - mu) * (u - mu), axis=0, keepdims=True)
        r = (u - mu) * lax.rsqrt(var + 1e-5) * g_ref[...] + be_ref[...]
        return jnp.maximum(r, 0.0)

    h = layer(h, W1_ref, b1_ref, g1_ref, be1_ref)
    h = layer(h, W2_ref, b2_ref, g2_ref, be2_ref)
    out_ref[...] = h


def kernel(p, q, x, W1, b1, g1, be1, W2, b2, g2, be2):
    pT = jnp.swapaxes(p, 1, 2)
    xT = jnp.swapaxes(x, 1, 2)

    hT = pl.pallas_call(
        _knn_interp_body,
        grid=(B, NB),
        in_specs=[
            pl.BlockSpec((1, TM, 3), lambda b, i: (b, i, 0)),
            pl.BlockSpec((1, 3, N), lambda b, i: (b, 0, 0)),
            pl.BlockSpec((1, N, C_IN), lambda b, i: (b, 0, 0)),
        ],
        out_specs=pl.BlockSpec((1, TM, C_IN), lambda b, i: (b, i, 0)),
        out_shape=jax.ShapeDtypeStruct((B, M, C_IN), jnp.float32),
    )(q, pT, xT)

    out = pl.pallas_call(
        _mlp_body,
        out_shape=jax.ShapeDtypeStruct((B * M, C2), jnp.float32),
    )(hT.reshape(B * M, C_IN), W1, b1[None, :], g1[None, :], be1[None, :],
      W2, b2[None, :], g2[None, :], be2[None, :])

    h = jnp.swapaxes(out.reshape(B, M, C2), 1, 2)
    return (q, h)
